# baseline (speedup 1.0000x reference)
import functools

import jax
import jax.numpy as jnp
from jax import lax
from jax.experimental import pallas as pl
from jax.experimental.pallas import tpu as pltpu
from jax.experimental.pallas import tpu_sc as plsc

N_NODES = 10000
D_IN = 128
D_HID = 16
D_OUT = 128

NC = 2
NS = 16
LANES = 16
CHUNK = 128
N_ACC = 10240
ROWS_PER_SUB = N_ACC // NS

_mesh = plsc.VectorSubcoreMesh(core_axis_name="c", subcore_axis_name="s")
_sc_params = pltpu.CompilerParams(use_tc_tiling_on_sc=False)


def _make_seg_sum(n_chunks, with_counts):
    out_types = [jax.ShapeDtypeStruct((NC, N_ACC, LANES), jnp.float32)]
    scratch = [
        pltpu.VMEM((n_chunks, CHUNK), jnp.int32),
        pltpu.VMEM((n_chunks, CHUNK), jnp.int32),
        pltpu.VMEM((CHUNK, LANES), jnp.float32),
        pltpu.VMEM((CHUNK, LANES), jnp.float32),
        pltpu.VMEM((CHUNK, LANES), jnp.float32),
        pltpu.VMEM_SHARED((N_ACC, LANES), jnp.float32),
        pltpu.VMEM_SHARED((N_ACC, LANES), jnp.float32),
        pltpu.SemaphoreType.DMA,
        pltpu.SemaphoreType.DMA,
        pltpu.SemaphoreType.DMA,
        pltpu.SemaphoreType.DMA,
        pltpu.SemaphoreType.DMA,
        pltpu.SemaphoreType.DMA,
    ]
    assert n_chunks % 2 == 0
    if with_counts:
        out_types.append(jax.ShapeDtypeStruct((NC, N_ACC, LANES), jnp.float32))

        @functools.partial(pl.kernel, out_type=out_types, mesh=_mesh,
                           scratch_types=scratch,
                           compiler_params=_sc_params)
        def seg(feat_hbm, src_hbm, dst_hbm, zero_hbm, ones_hbm,
                out_hbm, cnt_hbm,
                src_v, dst_v, rows0, rows1, ones_v, acc_sh, cnt_sh,
                g0, g1, s0, s1, t0, t1):
            c = lax.axis_index("c")
            s = lax.axis_index("s")
            r0 = s * ROWS_PER_SUB
            rows = pl.ds(r0, ROWS_PER_SUB)
            pltpu.sync_copy(zero_hbm.at[rows], acc_sh.at[rows])
            pltpu.sync_copy(zero_hbm.at[rows], cnt_sh.at[rows])
            pltpu.sync_copy(src_hbm.at[c].at[s], src_v)
            pltpu.sync_copy(dst_hbm.at[c].at[s], dst_v)
            pltpu.sync_copy(ones_hbm, ones_v)
            plsc.subcore_barrier()

            @pl.loop(0, n_chunks // 2)
            def _(i):
                j0 = 2 * i
                j1 = j0 + 1
                cg0 = pltpu.async_copy(feat_hbm.at[src_v.at[j0]], rows0, g0)
                cg1 = pltpu.async_copy(feat_hbm.at[src_v.at[j1]], rows1, g1)
                cg0.wait()
                cs0 = pltpu.async_copy(rows0, acc_sh.at[dst_v.at[j0]], s0,
                                       add=True)
                ct0 = pltpu.async_copy(ones_v, cnt_sh.at[dst_v.at[j0]], t0,
                                       add=True)
                cg1.wait()
                cs1 = pltpu.async_copy(rows1, acc_sh.at[dst_v.at[j1]], s1,
                                       add=True)
                ct1 = pltpu.async_copy(ones_v, cnt_sh.at[dst_v.at[j1]], t1,
                                       add=True)
                cs0.wait()
                cs1.wait()
                ct0.wait()
                ct1.wait()

            plsc.subcore_barrier()
            pltpu.sync_copy(acc_sh.at[rows], out_hbm.at[c].at[rows])
            pltpu.sync_copy(cnt_sh.at[rows], cnt_hbm.at[c].at[rows])

        return seg

    @functools.partial(pl.kernel, out_type=out_types, mesh=_mesh,
                       scratch_types=scratch,
                       compiler_params=_sc_params)
    def seg(feat_hbm, src_hbm, dst_hbm, zero_hbm, ones_hbm,
            out_hbm,
            src_v, dst_v, rows0, rows1, ones_v, acc_sh, cnt_sh,
            g0, g1, s0, s1, t0, t1):
        c = lax.axis_index("c")
        s = lax.axis_index("s")
        r0 = s * ROWS_PER_SUB
        rows = pl.ds(r0, ROWS_PER_SUB)
        pltpu.sync_copy(zero_hbm.at[rows], acc_sh.at[rows])
        pltpu.sync_copy(src_hbm.at[c].at[s], src_v)
        pltpu.sync_copy(dst_hbm.at[c].at[s], dst_v)
        plsc.subcore_barrier()

        @pl.loop(0, n_chunks // 2)
        def _(i):
            j0 = 2 * i
            j1 = j0 + 1
            cg0 = pltpu.async_copy(feat_hbm.at[src_v.at[j0]], rows0, g0)
            cg1 = pltpu.async_copy(feat_hbm.at[src_v.at[j1]], rows1, g1)
            cg0.wait()
            cs0 = pltpu.async_copy(rows0, acc_sh.at[dst_v.at[j0]], s0,
                                   add=True)
            cg1.wait()
            cs1 = pltpu.async_copy(rows1, acc_sh.at[dst_v.at[j1]], s1,
                                   add=True)
            cs0.wait()
            cs1.wait()

        plsc.subcore_barrier()
        pltpu.sync_copy(acc_sh.at[rows], out_hbm.at[c].at[rows])

    return seg


def _mm_in_body(x_ref, wl_ref, wr_ref, y1_ref, xr_ref):
    x = x_ref[...]
    y1_ref[...] = jnp.dot(x, wl_ref[...], preferred_element_type=jnp.float32)
    xr_ref[...] = jnp.dot(x, wr_ref[...], preferred_element_type=jnp.float32)


def _h_body(p_ref, cp_ref, xr_ref, b1_ref, h_ref, cnt_ref):
    cnt = cp_ref[0] + cp_ref[1]
    mean = (p_ref[0] + p_ref[1]) / jnp.maximum(cnt, 1.0)
    h_ref[...] = jnp.maximum(mean + b1_ref[...] + xr_ref[...], 0.0)
    cnt_ref[...] = cnt


def _out_body(p2_ref, cnt_ref, h_ref, w2l_ref, b2_ref, w2r_ref, o_ref):
    mean2 = (p2_ref[0] + p2_ref[1]) / jnp.maximum(cnt_ref[...], 1.0)
    o_ref[...] = (
        jnp.dot(mean2, w2l_ref[...], preferred_element_type=jnp.float32)
        + b2_ref[...]
        + jnp.dot(h_ref[...], w2r_ref[...], preferred_element_type=jnp.float32)
    )


@jax.jit
def _impl(x, edge_index, W1l, b1, W1r, W2l, b2, W2r):
    src = edge_index[0].astype(jnp.int32)
    dst = edge_index[1].astype(jnp.int32)
    e = src.shape[0]
    n_chunks = -(-e // (NC * NS * CHUNK))
    if n_chunks % 2:
        n_chunks += 1
    e_pad = NC * NS * n_chunks * CHUNK
    src3 = jnp.concatenate(
        [src, jnp.zeros((e_pad - e,), jnp.int32)]).reshape(
            NC, NS, n_chunks, CHUNK)
    dst3 = jnp.concatenate(
        [dst, jnp.full((e_pad - e,), N_NODES, jnp.int32)]).reshape(
            NC, NS, n_chunks, CHUNK)
    xp = jnp.zeros((N_ACC, D_IN), jnp.float32).at[:N_NODES].set(x)
    zero = jnp.zeros((N_ACC, LANES), jnp.float32)
    ones = jnp.ones((CHUNK, LANES), jnp.float32)

    y1, xr = pl.pallas_call(
        _mm_in_body,
        out_shape=[jax.ShapeDtypeStruct((N_ACC, D_HID), jnp.float32)] * 2,
    )(xp, W1l, W1r)

    seg_c = _make_seg_sum(n_chunks, with_counts=True)
    p1, c1 = seg_c(y1, src3, dst3, zero, ones)

    h, cnt = pl.pallas_call(
        _h_body,
        out_shape=[jax.ShapeDtypeStruct((N_ACC, D_HID), jnp.float32)] * 2,
    )(p1, c1, xr, b1.reshape(1, D_HID))

    seg_p = _make_seg_sum(n_chunks, with_counts=False)
    (p2,) = seg_p(h, src3, dst3, zero, ones)

    out = pl.pallas_call(
        _out_body,
        out_shape=jax.ShapeDtypeStruct((N_ACC, D_OUT), jnp.float32),
    )(p2, cnt, h, W2l, b2.reshape(1, D_OUT), W2r)
    return out[:N_NODES]


def kernel(x, edge_index, W1l, b1, W1r, W2l, b2, W2r):
    return _impl(x, edge_index, W1l, b1, W1r, W2l, b2, W2r)


# speedup vs baseline: 1.2179x; 1.2179x over previous
import functools

import jax
import jax.numpy as jnp
from jax import lax
from jax.experimental import pallas as pl
from jax.experimental.pallas import tpu as pltpu
from jax.experimental.pallas import tpu_sc as plsc

N_NODES = 10000
D_IN = 128
D_HID = 16
D_OUT = 128

NC = 2
NS = 16
LANES = 16
GCHUNK = 512
SCHUNK = 512
SPG = GCHUNK // SCHUNK
N_ACC = 10240
ROWS_PER_SUB = N_ACC // NS

_mesh = plsc.VectorSubcoreMesh(core_axis_name="c", subcore_axis_name="s")
_sc_params = pltpu.CompilerParams(use_tc_tiling_on_sc=False)


def _make_seg_sum(n_g, with_counts):
    out_types = [jax.ShapeDtypeStruct((NC, N_ACC, LANES), jnp.float32)]
    if with_counts:
        out_types.append(jax.ShapeDtypeStruct((NC, N_ACC, LANES), jnp.float32))
    n_s = n_g * SPG
    scratch = [
        pltpu.VMEM((n_g, GCHUNK), jnp.int32),
        pltpu.VMEM((n_s, SCHUNK), jnp.int32),
        pltpu.VMEM((GCHUNK, LANES), jnp.float32),
        pltpu.VMEM((GCHUNK, LANES), jnp.float32),
        pltpu.VMEM((SCHUNK, LANES), jnp.float32),
        pltpu.VMEM_SHARED((N_ACC, LANES), jnp.float32),
        pltpu.VMEM_SHARED((N_ACC, LANES), jnp.float32),
    ] + [pltpu.SemaphoreType.DMA] * 6
    assert n_g % 2 == 0

    def body(*refs):
        if with_counts:
            (feat_hbm, src_hbm, dst_hbm, zero_hbm, ones_hbm, out_hbm, cnt_hbm,
             src_v, dst_v, rows0, rows1, ones_v, acc_sh, cnt_sh,
             g0, g1, s0, s1, t0, t1) = refs
        else:
            (feat_hbm, src_hbm, dst_hbm, zero_hbm, ones_hbm, out_hbm,
             src_v, dst_v, rows0, rows1, ones_v, acc_sh, cnt_sh,
             g0, g1, s0, s1, t0, t1) = refs
            cnt_hbm = None
        c = lax.axis_index("c")
        s = lax.axis_index("s")
        rows = pl.ds(s * ROWS_PER_SUB, ROWS_PER_SUB)
        pltpu.sync_copy(zero_hbm.at[rows], acc_sh.at[rows])
        if with_counts:
            pltpu.sync_copy(zero_hbm.at[rows], cnt_sh.at[rows])
            pltpu.sync_copy(ones_hbm, ones_v)
        pltpu.sync_copy(src_hbm.at[c].at[s], src_v)
        pltpu.sync_copy(dst_hbm.at[c].at[s], dst_v)
        plsc.subcore_barrier()

        def scat(j, buf, sem, tsem):
            ds_ = []
            for k in range(SPG):
                jj = j * SPG + k
                ds_.append(pltpu.async_copy(
                    buf.at[pl.ds(k * SCHUNK, SCHUNK)],
                    acc_sh.at[dst_v.at[jj]], sem, add=True))
                if with_counts:
                    ds_.append(pltpu.async_copy(
                        ones_v, cnt_sh.at[dst_v.at[jj]], tsem, add=True))
            return ds_

        @pl.loop(0, n_g // 2)
        def _(i):
            j0 = 2 * i
            j1 = j0 + 1
            cg0 = pltpu.async_copy(feat_hbm.at[src_v.at[j0]], rows0, g0)
            cg1 = pltpu.async_copy(feat_hbm.at[src_v.at[j1]], rows1, g1)
            cg0.wait()
            d0 = scat(j0, rows0, s0, t0)
            cg1.wait()
            d1 = scat(j1, rows1, s1, t1)
            for d in d0 + d1:
                d.wait()

        plsc.subcore_barrier()
        pltpu.sync_copy(acc_sh.at[rows], out_hbm.at[c].at[rows])
        if with_counts:
            pltpu.sync_copy(cnt_sh.at[rows], cnt_hbm.at[c].at[rows])

    return pl.kernel(body, out_type=out_types, mesh=_mesh,
                     scratch_types=scratch, compiler_params=_sc_params)


def _mm_in_body(x_ref, wl_ref, wr_ref, y1_ref, xr_ref):
    x = x_ref[...]
    y1_ref[...] = jnp.dot(x, wl_ref[...], preferred_element_type=jnp.float32)
    xr_ref[...] = jnp.dot(x, wr_ref[...], preferred_element_type=jnp.float32)


def _h_body(p_ref, cp_ref, xr_ref, b1_ref, h_ref, cnt_ref):
    cnt = cp_ref[0] + cp_ref[1]
    mean = (p_ref[0] + p_ref[1]) / jnp.maximum(cnt, 1.0)
    h_ref[...] = jnp.maximum(mean + b1_ref[...] + xr_ref[...], 0.0)
    cnt_ref[...] = cnt


def _out_body(p2_ref, cnt_ref, h_ref, w2l_ref, b2_ref, w2r_ref, o_ref):
    mean2 = (p2_ref[0] + p2_ref[1]) / jnp.maximum(cnt_ref[...], 1.0)
    o_ref[...] = (
        jnp.dot(mean2, w2l_ref[...], preferred_element_type=jnp.float32)
        + b2_ref[...]
        + jnp.dot(h_ref[...], w2r_ref[...], preferred_element_type=jnp.float32)
    )


@jax.jit
def _impl(x, edge_index, W1l, b1, W1r, W2l, b2, W2r):
    src = edge_index[0].astype(jnp.int32)
    dst = edge_index[1].astype(jnp.int32)
    e = src.shape[0]
    n_g = -(-e // (NC * NS * GCHUNK))
    if n_g % 2:
        n_g += 1
    e_pad = NC * NS * n_g * GCHUNK
    src3 = jnp.concatenate(
        [src, jnp.zeros((e_pad - e,), jnp.int32)]).reshape(
            NC, NS, n_g, GCHUNK)
    dst3 = jnp.concatenate(
        [dst, jnp.full((e_pad - e,), N_NODES, jnp.int32)]).reshape(
            NC, NS, n_g * SPG, SCHUNK)
    xp = jnp.zeros((N_ACC, D_IN), jnp.float32).at[:N_NODES].set(x)
    zero = jnp.zeros((N_ACC, LANES), jnp.float32)
    ones = jnp.ones((SCHUNK, LANES), jnp.float32)

    y1, xr = pl.pallas_call(
        _mm_in_body,
        out_shape=[jax.ShapeDtypeStruct((N_ACC, D_HID), jnp.float32)] * 2,
    )(xp, W1l, W1r)

    seg_c = _make_seg_sum(n_g, with_counts=True)
    p1, c1 = seg_c(y1, src3, dst3, zero, ones)

    h, cnt = pl.pallas_call(
        _h_body,
        out_shape=[jax.ShapeDtypeStruct((N_ACC, D_HID), jnp.float32)] * 2,
    )(p1, c1, xr, b1.reshape(1, D_HID))

    seg_p = _make_seg_sum(n_g, with_counts=False)
    (p2,) = seg_p(h, src3, dst3, zero, ones)

    out = pl.pallas_call(
        _out_body,
        out_shape=jax.ShapeDtypeStruct((N_ACC, D_OUT), jnp.float32),
    )(p2, cnt, h, W2l, b2.reshape(1, D_OUT), W2r)
    return out[:N_NODES]


def kernel(x, edge_index, W1l, b1, W1r, W2l, b2, W2r):
    return _impl(x, edge_index, W1l, b1, W1r, W2l, b2, W2r)


# speedup vs baseline: 2.1319x; 1.7505x over previous
import functools

import jax
import jax.numpy as jnp
from jax import lax
from jax.experimental import pallas as pl
from jax.experimental.pallas import tpu as pltpu
from jax.experimental.pallas import tpu_sc as plsc

N_NODES = 10000
D_IN = 128
D_HID = 16
D_OUT = 128

NC = 2
NS = 16
LANES = 16
GCHUNK = 512
NBUF = 8
LOOKAHEAD = 4
HBLK = 320
N_ACC = 10240
ROWS_PER_SUB = N_ACC // NS

_mesh = plsc.VectorSubcoreMesh(core_axis_name="c", subcore_axis_name="s")
_sc_params = pltpu.CompilerParams(use_tc_tiling_on_sc=False)


def _make_seg_sum(n_g, with_counts):
    out_types = [jax.ShapeDtypeStruct((NC, N_ACC, LANES), jnp.float32)]
    if with_counts:
        out_types.append(jax.ShapeDtypeStruct((NC, N_ACC, LANES), jnp.float32))
    scratch = (
        [
            pltpu.VMEM((n_g, GCHUNK), jnp.int32),
            pltpu.VMEM((n_g, GCHUNK), jnp.int32),
            pltpu.VMEM((GCHUNK, LANES), jnp.float32),
            pltpu.VMEM_SHARED((N_ACC, LANES), jnp.float32),
            pltpu.VMEM_SHARED((N_ACC, LANES), jnp.float32),
        ]
        + [pltpu.VMEM((GCHUNK, LANES), jnp.float32)] * NBUF
        + [pltpu.SemaphoreType.DMA] * (2 * NBUF + 1)
    )

    def body(*refs):
        if with_counts:
            (feat_hbm, src_hbm, dst_hbm, zero_hbm, ones_hbm, out_hbm, cnt_hbm,
             src_v, dst_v, ones_v, acc_sh, cnt_sh, *rest) = refs
        else:
            (feat_hbm, src_hbm, dst_hbm, zero_hbm, ones_hbm, out_hbm,
             src_v, dst_v, ones_v, acc_sh, cnt_sh, *rest) = refs
            cnt_hbm = None
        bufs = rest[:NBUF]
        gsem = rest[NBUF:2 * NBUF]
        ssem = rest[2 * NBUF:3 * NBUF]
        tsem = rest[3 * NBUF]
        c = lax.axis_index("c")
        s = lax.axis_index("s")
        rows = pl.ds(s * ROWS_PER_SUB, ROWS_PER_SUB)
        pltpu.sync_copy(zero_hbm.at[rows], acc_sh.at[rows])
        if with_counts:
            pltpu.sync_copy(zero_hbm.at[rows], cnt_sh.at[rows])
            pltpu.sync_copy(ones_hbm, ones_v)
        pltpu.sync_copy(src_hbm.at[c].at[s], src_v)
        pltpu.sync_copy(dst_hbm.at[c].at[s], dst_v)
        plsc.subcore_barrier()

        gds = [None] * n_g
        sds = [None] * n_g
        cds = []
        for j in range(min(LOOKAHEAD, n_g)):
            gds[j] = pltpu.async_copy(feat_hbm.at[src_v.at[j]],
                                      bufs[j % NBUF], gsem[j % NBUF])
        for j in range(n_g):
            nj = j + LOOKAHEAD
            if nj < n_g:
                if nj >= NBUF:
                    sds[nj - NBUF].wait()
                gds[nj] = pltpu.async_copy(feat_hbm.at[src_v.at[nj]],
                                           bufs[nj % NBUF], gsem[nj % NBUF])
            gds[j].wait()
            sds[j] = pltpu.async_copy(bufs[j % NBUF], acc_sh.at[dst_v.at[j]],
                                      ssem[j % NBUF], add=True)
            if with_counts:
                cds.append(pltpu.async_copy(ones_v, cnt_sh.at[dst_v.at[j]],
                                            tsem, add=True))
        for j in range(max(0, n_g - NBUF), n_g):
            sds[j].wait()
        for d in cds:
            d.wait()

        plsc.subcore_barrier()
        pltpu.sync_copy(acc_sh.at[rows], out_hbm.at[c].at[rows])
        if with_counts:
            pltpu.sync_copy(cnt_sh.at[rows], cnt_hbm.at[c].at[rows])

    return pl.kernel(body, out_type=out_types, mesh=_mesh,
                     scratch_types=scratch, compiler_params=_sc_params)


def _make_seg_h(n_g):
    out_types = [jax.ShapeDtypeStruct((NC, N_ACC, LANES), jnp.float32),
                 jax.ShapeDtypeStruct((NC, N_ACC, LANES), jnp.float32)]
    scratch = (
        [
            pltpu.VMEM((n_g, GCHUNK), jnp.int32),
            pltpu.VMEM((n_g, GCHUNK), jnp.int32),
            pltpu.VMEM_SHARED((N_ACC, LANES), jnp.float32),
        ]
        + [pltpu.VMEM((GCHUNK, LANES), jnp.float32)] * NBUF
        + [pltpu.SemaphoreType.DMA] * (2 * NBUF)
    )

    def body(p1_hbm, c1_hbm, xb_hbm, src_hbm, dst_hbm, zero_hbm,
             out_hbm, h_hbm, src_v, dst_v, acc_sh, *rest):
        bufs = rest[:NBUF]
        gsem = rest[NBUF:2 * NBUF]
        ssem = rest[2 * NBUF:3 * NBUF]
        c = lax.axis_index("c")
        s = lax.axis_index("s")
        rows = pl.ds(s * ROWS_PER_SUB, ROWS_PER_SUB)
        pltpu.sync_copy(zero_hbm.at[rows], acc_sh.at[rows])
        pltpu.sync_copy(src_hbm.at[c].at[s], src_v)
        pltpu.sync_copy(dst_hbm.at[c].at[s], dst_v)

        blk = pl.ds(0, HBLK)
        for b in range(ROWS_PER_SUB // HBLK):
            rb = pl.ds(s * ROWS_PER_SUB + b * HBLK, HBLK)
            pltpu.sync_copy(p1_hbm.at[0].at[rb], bufs[0].at[blk])
            pltpu.sync_copy(p1_hbm.at[1].at[rb], bufs[1].at[blk])
            pltpu.sync_copy(c1_hbm.at[0].at[rb], bufs[2].at[blk])
            pltpu.sync_copy(c1_hbm.at[1].at[rb], bufs[3].at[blk])
            pltpu.sync_copy(xb_hbm.at[rb], bufs[4].at[blk])

            @pl.loop(0, HBLK)
            def _(r):
                rr = (pl.ds(r, 1), pl.ds(0, LANES))
                p = bufs[0][rr] + bufs[1][rr]
                cn = jnp.maximum(bufs[2][rr] + bufs[3][rr], 1.0)
                bufs[5][rr] = jnp.maximum(p / cn + bufs[4][rr], 0.0)

            pltpu.sync_copy(bufs[5].at[blk], h_hbm.at[c].at[rb])
        plsc.subcore_barrier()

        feat_hbm = h_hbm.at[c]
        gds = [None] * n_g
        sds = [None] * n_g
        for j in range(min(LOOKAHEAD, n_g)):
            gds[j] = pltpu.async_copy(feat_hbm.at[src_v.at[j]],
                                      bufs[j % NBUF], gsem[j % NBUF])
        for j in range(n_g):
            nj = j + LOOKAHEAD
            if nj < n_g:
                if nj >= NBUF:
                    sds[nj - NBUF].wait()
                gds[nj] = pltpu.async_copy(feat_hbm.at[src_v.at[nj]],
                                           bufs[nj % NBUF], gsem[nj % NBUF])
            gds[j].wait()
            sds[j] = pltpu.async_copy(bufs[j % NBUF], acc_sh.at[dst_v.at[j]],
                                      ssem[j % NBUF], add=True)
        for j in range(max(0, n_g - NBUF), n_g):
            sds[j].wait()

        plsc.subcore_barrier()
        pltpu.sync_copy(acc_sh.at[rows], out_hbm.at[c].at[rows])

    return pl.kernel(body, out_type=out_types, mesh=_mesh,
                     scratch_types=scratch, compiler_params=_sc_params)


def _mm_in_body(x_ref, wl_ref, wr_ref, b1_ref, y1_ref, xb_ref):
    x = x_ref[...]
    y1_ref[...] = jnp.dot(x, wl_ref[...], preferred_element_type=jnp.float32)
    xb_ref[...] = jnp.dot(
        x, wr_ref[...], preferred_element_type=jnp.float32) + b1_ref[...]


def _h_body(p_ref, cp_ref, xr_ref, b1_ref, h_ref, cnt_ref):
    cnt = cp_ref[0] + cp_ref[1]
    mean = (p_ref[0] + p_ref[1]) / jnp.maximum(cnt, 1.0)
    h_ref[...] = jnp.maximum(mean + b1_ref[...] + xr_ref[...], 0.0)
    cnt_ref[...] = cnt


def _out_body(p2_ref, c1_ref, h_ref, w2l_ref, b2_ref, w2r_ref, o_ref):
    cnt = c1_ref[0] + c1_ref[1]
    mean2 = (p2_ref[0] + p2_ref[1]) / jnp.maximum(cnt, 1.0)
    o_ref[...] = (
        jnp.dot(mean2, w2l_ref[...], preferred_element_type=jnp.float32)
        + b2_ref[...]
        + jnp.dot(h_ref[0], w2r_ref[...], preferred_element_type=jnp.float32)
    )


@jax.jit
def _impl(x, edge_index, W1l, b1, W1r, W2l, b2, W2r):
    src = edge_index[0].astype(jnp.int32)
    dst = edge_index[1].astype(jnp.int32)
    e = src.shape[0]
    n_g = -(-e // (NC * NS * GCHUNK))
    if n_g % 2:
        n_g += 1
    e_pad = NC * NS * n_g * GCHUNK
    pad_i = jnp.arange(e_pad - e, dtype=jnp.int32)
    src3 = jnp.concatenate(
        [src, pad_i % N_NODES]).reshape(NC, NS, n_g, GCHUNK)
    dst3 = jnp.concatenate(
        [dst, N_NODES + pad_i % (N_ACC - N_NODES)]).reshape(
            NC, NS, n_g, GCHUNK)
    xp = jnp.zeros((N_ACC, D_IN), jnp.float32).at[:N_NODES].set(x)
    zero = jnp.zeros((N_ACC, LANES), jnp.float32)
    ones = jnp.ones((GCHUNK, LANES), jnp.float32)

    y1, xb = pl.pallas_call(
        _mm_in_body,
        out_shape=[jax.ShapeDtypeStruct((N_ACC, D_HID), jnp.float32)] * 2,
    )(xp, W1l, W1r, b1.reshape(1, D_HID))

    seg_c = _make_seg_sum(n_g, with_counts=True)
    p1, c1 = seg_c(y1, src3, dst3, zero, ones)

    seg_h = _make_seg_h(n_g)
    p2, h2 = seg_h(p1, c1, xb, src3, dst3, zero)

    out = pl.pallas_call(
        _out_body,
        out_shape=jax.ShapeDtypeStruct((N_ACC, D_OUT), jnp.float32),
    )(p2, c1, h2, W2l, b2.reshape(1, D_OUT), W2r)
    return out[:N_NODES]


def kernel(x, edge_index, W1l, b1, W1r, W2l, b2, W2r):
    return _impl(x, edge_index, W1l, b1, W1r, W2l, b2, W2r)
